# baseline (speedup 1.0000x reference)
import jax
import jax.numpy as jnp
from jax.experimental import pallas as pl
from jax.experimental.pallas import tpu as pltpu


def _chamfer_kernel(src_ref, tgt_t_ref, out_src_ref, out_dst_ref):
    b = pl.program_id(0)

    s = src_ref[0]
    t = tgt_t_ref[0]

    sn = jnp.sum(s * s, axis=1, keepdims=True)
    tn = jnp.sum(t * t, axis=0, keepdims=True)

    cross = jax.lax.dot_general(
        s, t, (((1,), (0,)), ((), ())),
        preferred_element_type=jnp.float32,
        precision=jax.lax.Precision.HIGHEST,
    )

    dist = (sn + tn) - (cross + cross)

    row_min = jnp.min(dist, axis=1, keepdims=True)
    col_min = jnp.min(dist, axis=0, keepdims=True)

    src_sum = jnp.sum(row_min, axis=0, keepdims=True)
    dst_sum = jnp.sum(col_min, axis=1, keepdims=True)

    @pl.when(b == 0)
    def _init():
        out_src_ref[...] = jnp.zeros_like(out_src_ref)
        out_dst_ref[...] = jnp.zeros_like(out_dst_ref)

    out_src_ref[...] += src_sum
    out_dst_ref[...] += dst_sum


def kernel(source, target):
    B, N, _ = source.shape
    M = target.shape[1]

    target_t = jnp.swapaxes(target, 1, 2)

    out_src, out_dst = pl.pallas_call(
        _chamfer_kernel,
        grid=(B,),
        in_specs=[
            pl.BlockSpec((1, N, 3), lambda b: (b, 0, 0)),
            pl.BlockSpec((1, 3, M), lambda b: (b, 0, 0)),
        ],
        out_specs=[
            pl.BlockSpec((1, 1), lambda b: (0, 0)),
            pl.BlockSpec((1, 1), lambda b: (0, 0)),
        ],
        out_shape=[
            jax.ShapeDtypeStruct((1, 1), jnp.float32),
            jax.ShapeDtypeStruct((1, 1), jnp.float32),
        ],
    )(source, target_t)

    loss_src = out_src[0, 0] / (B * N)
    loss_dst = out_dst[0, 0] / (B * M)
    return (loss_src, loss_dst)


# speedup vs baseline: 1.8624x; 1.8624x over previous
import jax
import jax.numpy as jnp
from jax.experimental import pallas as pl
from jax.experimental.pallas import tpu as pltpu


def _chamfer_kernel(src_ref, tgt_t_ref, out_src_ref, out_dst_ref):
    b = pl.program_id(0)

    s = src_ref[0]
    t = tgt_t_ref[0]

    sx = s[:, 0:1]
    sy = s[:, 1:2]
    sz = s[:, 2:3]
    tx = t[0:1, :]
    ty = t[1:2, :]
    tz = t[2:3, :]

    dx = sx - tx
    dy = sy - ty
    dz = sz - tz
    dist = dx * dx + dy * dy + dz * dz

    row_min = jnp.min(dist, axis=1, keepdims=True)
    col_min = jnp.min(dist, axis=0, keepdims=True)

    src_sum = jnp.sum(row_min, axis=0, keepdims=True)
    dst_sum = jnp.sum(col_min, axis=1, keepdims=True)

    @pl.when(b == 0)
    def _init():
        out_src_ref[...] = jnp.zeros_like(out_src_ref)
        out_dst_ref[...] = jnp.zeros_like(out_dst_ref)

    out_src_ref[...] += src_sum
    out_dst_ref[...] += dst_sum


def kernel(source, target):
    B, N, _ = source.shape
    M = target.shape[1]

    target_t = jnp.swapaxes(target, 1, 2)

    out_src, out_dst = pl.pallas_call(
        _chamfer_kernel,
        grid=(B,),
        in_specs=[
            pl.BlockSpec((1, N, 3), lambda b: (b, 0, 0)),
            pl.BlockSpec((1, 3, M), lambda b: (b, 0, 0)),
        ],
        out_specs=[
            pl.BlockSpec((1, 1), lambda b: (0, 0)),
            pl.BlockSpec((1, 1), lambda b: (0, 0)),
        ],
        out_shape=[
            jax.ShapeDtypeStruct((1, 1), jnp.float32),
            jax.ShapeDtypeStruct((1, 1), jnp.float32),
        ],
    )(source, target_t)

    loss_src = out_src[0, 0] / (B * N)
    loss_dst = out_dst[0, 0] / (B * M)
    return (loss_src, loss_dst)


# speedup vs baseline: 1.9960x; 1.0717x over previous
import jax
import jax.numpy as jnp
from jax.experimental import pallas as pl
from jax.experimental.pallas import tpu as pltpu


def _chamfer_kernel(src_ref, tgt_ref, out_src_ref, out_dst_ref):
    b = pl.program_id(0)
    nb = pl.num_programs(0)

    s = src_ref[0]
    t = tgt_ref[0]
    tt = t.T

    sx = s[:, 0:1]
    sy = s[:, 1:2]
    sz = s[:, 2:3]
    tx = tt[0:1, :]
    ty = tt[1:2, :]
    tz = tt[2:3, :]

    dx = sx - tx
    dy = sy - ty
    dz = sz - tz
    dist = dx * dx + dy * dy + dz * dz

    row_min = jnp.min(dist, axis=1, keepdims=True)
    col_min = jnp.min(dist, axis=0, keepdims=True)

    src_sum = jnp.sum(row_min, axis=0, keepdims=True)
    dst_sum = jnp.sum(col_min, axis=1, keepdims=True)

    @pl.when(b == 0)
    def _init():
        out_src_ref[...] = jnp.zeros_like(out_src_ref)
        out_dst_ref[...] = jnp.zeros_like(out_dst_ref)

    out_src_ref[...] += src_sum
    out_dst_ref[...] += dst_sum

    @pl.when(b == nb - 1)
    def _finish():
        n_src = jnp.float32(src_ref.shape[1] * nb)
        n_dst = jnp.float32(tgt_ref.shape[1] * nb)
        out_src_ref[...] = out_src_ref[...] * (1.0 / n_src)
        out_dst_ref[...] = out_dst_ref[...] * (1.0 / n_dst)


def kernel(source, target):
    B, N, _ = source.shape
    M = target.shape[1]

    out_src, out_dst = pl.pallas_call(
        _chamfer_kernel,
        grid=(B,),
        in_specs=[
            pl.BlockSpec((1, N, 3), lambda b: (b, 0, 0)),
            pl.BlockSpec((1, M, 3), lambda b: (b, 0, 0)),
        ],
        out_specs=[
            pl.BlockSpec((1, 1), lambda b: (0, 0)),
            pl.BlockSpec((1, 1), lambda b: (0, 0)),
        ],
        out_shape=[
            jax.ShapeDtypeStruct((1, 1), jnp.float32),
            jax.ShapeDtypeStruct((1, 1), jnp.float32),
        ],
    )(source, target)

    return (out_src[0, 0], out_dst[0, 0])
